# baseline (speedup 1.0000x reference)
import functools

import jax
import jax.numpy as jnp
from jax import lax
from jax.experimental import pallas as pl
from jax.experimental.pallas import tpu as pltpu
from jax.experimental.pallas import tpu_sc as plsc

N_NODES = 10000
D = 128
HD = D // 2
E = 320000

NC = 2
NS = 16
C = 128
NCH = 160
E_PAD = NS * NCH * C
N_PAD = 10240
RPT = N_PAD // NS
PIECE = 80
NPIECE = RPT // PIECE

_sc_mesh = plsc.VectorSubcoreMesh(core_axis_name="c", subcore_axis_name="s")
_sc_params = pltpu.CompilerParams(use_tc_tiling_on_sc=False,
                                  needs_layout_passes=False)


def _rsqrt16(d):
    i = plsc.bitcast(d, jnp.int32)
    y = plsc.bitcast(jnp.int32(0x5F3759DF) - (i >> 1), jnp.float32)
    for _ in range(3):
        y = y * (1.5 - 0.5 * d * y * y)
    return jnp.where(d > 0.5, y, 0.0)


@functools.partial(
    pl.kernel,
    out_type=(
        jax.ShapeDtypeStruct((NC, N_PAD), jnp.float32),
        jax.ShapeDtypeStruct((NC, N_NODES, HD), jnp.float32),
    ),
    mesh=_sc_mesh,
    scratch_types=[
        pltpu.VMEM((NCH, C), jnp.int32),
        pltpu.VMEM((C,), jnp.float32),
        pltpu.VMEM((RPT,), jnp.float32),
        pltpu.VMEM((RPT,), jnp.float32),
        pltpu.VMEM((RPT,), jnp.float32),
        pltpu.VMEM((16,), jnp.float32),
        pltpu.VMEM((PIECE, D), jnp.float32),
        pltpu.VMEM((PIECE, HD), jnp.float32),
        pltpu.VMEM_SHARED((N_PAD,), jnp.float32),
    ],
    compiler_params=_sc_params,
)
def _norm_kernel(col_hbm, alpha_hbm, x_hbm, adis_hbm, xs_hbm,
                 col_v, ones_v, deg_v, dis_v, adis_v, alpha_v,
                 xp_v, xsp_v, deg_sh):
    c = lax.axis_index("c")
    s = lax.axis_index("s")
    one16 = jnp.ones((16,), jnp.float32)
    zero16 = jnp.zeros((16,), jnp.float32)
    for i in range(C // 16):
        ones_v[pl.ds(i * 16, 16)] = one16
    for i in range(RPT // 16):
        deg_v[pl.ds(i * 16, 16)] = zero16
    pltpu.sync_copy(deg_v, deg_sh.at[pl.ds(s * RPT, RPT)])
    pltpu.sync_copy(col_hbm.at[s], col_v)
    pltpu.sync_copy(alpha_hbm, alpha_v)
    plsc.subcore_barrier()

    def hist(j, _):
        pltpu.sync_copy(ones_v, deg_sh.at[col_v.at[j]], add=True)
        return ()

    lax.fori_loop(0, NCH, hist, ())
    plsc.subcore_barrier()

    pltpu.sync_copy(deg_sh.at[pl.ds(s * RPT, RPT)], deg_v)
    av = alpha_v[...]
    for i in range(RPT // 16):
        dis16 = _rsqrt16(deg_v[pl.ds(i * 16, 16)])
        dis_v[pl.ds(i * 16, 16)] = dis16
        adis_v[pl.ds(i * 16, 16)] = dis16 * av
    pltpu.sync_copy(adis_v, adis_hbm.at[c, pl.ds(s * RPT, RPT)])

    base_col = c * HD
    npieces = jnp.where(s == NS - 1, (N_NODES - (NS - 1) * RPT) // PIECE,
                        NPIECE)

    def prescale(p, _):
        r0 = s * RPT + p * PIECE
        pltpu.sync_copy(x_hbm.at[pl.ds(r0, PIECE)], xp_v)

        def groupscale(g, _):
            dis16 = dis_v[pl.ds(p * PIECE + g * 16, 16)]
            for l in range(16):
                i = g * 16 + l
                dval = dis16[l]
                for jj in range(HD // 16):
                    xsp_v[i, pl.ds(jj * 16, 16)] = (
                        xp_v[i, pl.ds(base_col + jj * 16, 16)] * dval)
            return ()

        lax.fori_loop(0, PIECE // 16, groupscale, ())
        pltpu.sync_copy(xsp_v, xs_hbm.at[c, pl.ds(r0, PIECE)])
        return ()

    lax.fori_loop(0, npieces, prescale, ())


@functools.partial(
    pl.kernel,
    out_type=jax.ShapeDtypeStruct((N_NODES, NC, HD), jnp.float32),
    mesh=_sc_mesh,
    scratch_types=[
        pltpu.VMEM((NCH, C), jnp.int32),
        pltpu.VMEM((NCH, C), jnp.int32),
        pltpu.VMEM((4, C, HD), jnp.float32),
        pltpu.VMEM((C, HD), jnp.float32),
        pltpu.VMEM((RPT,), jnp.float32),
        pltpu.VMEM((PIECE, HD), jnp.float32),
        pltpu.VMEM_SHARED((N_PAD, HD), jnp.float32),
        pltpu.SemaphoreType.DMA,
        pltpu.SemaphoreType.DMA,
    ],
    compiler_params=_sc_params,
)
def _agg_kernel(xs_hbm, row_hbm, col_hbm, adis_hbm, out_hbm,
                row_v, col_v, msg_v, zero_v, adis_v, stage_v, acc_sh,
                gsem, ssem):
    c = lax.axis_index("c")
    s = lax.axis_index("s")
    zero16 = jnp.zeros((16,), jnp.float32)

    def zbody(i, _):
        for jj in range(HD // 16):
            zero_v[i, pl.ds(jj * 16, 16)] = zero16
        return ()

    lax.fori_loop(0, C, zbody, ())
    for piece in range(RPT // C):
        pltpu.sync_copy(zero_v, acc_sh.at[pl.ds(s * RPT + piece * C, C)])
    pltpu.sync_copy(row_hbm.at[s], row_v)
    pltpu.sync_copy(col_hbm.at[s], col_v)
    pltpu.sync_copy(adis_hbm.at[c, pl.ds(s * RPT, RPT)], adis_v)
    plsc.subcore_barrier()

    xsc = xs_hbm.at[c]

    for b in range(3):
        pltpu.async_copy(xsc.at[row_v.at[b]], msg_v.at[b], gsem)

    def round_body(r, _):
        for b in range(4):
            k = 4 * r + b
            pltpu.make_async_copy(xsc.at[row_v.at[k]], msg_v.at[b], gsem).wait()
            pltpu.async_copy(msg_v.at[b], acc_sh.at[col_v.at[k]], ssem,
                             add=True)

            @pl.when(k >= 1)
            def _release_prev():
                pltpu.make_async_copy(msg_v.at[(b + 3) % 4],
                                      acc_sh.at[col_v.at[k - 1]], ssem).wait()

            @pl.when(k + 3 < NCH)
            def _launch_next():
                pltpu.async_copy(xsc.at[row_v.at[k + 3]],
                                 msg_v.at[(b + 3) % 4], gsem)
        return ()

    lax.fori_loop(0, NCH // 4, round_body, ())
    pltpu.make_async_copy(msg_v.at[3], acc_sh.at[col_v.at[NCH - 1]],
                          ssem).wait()
    plsc.subcore_barrier()

    npieces = jnp.where(s == NS - 1, (N_NODES - (NS - 1) * RPT) // PIECE,
                        NPIECE)

    def drain(p, _):
        r0 = s * RPT + p * PIECE
        pltpu.sync_copy(acc_sh.at[pl.ds(r0, PIECE)], stage_v)

        def groupscale(g, _):
            adis16 = adis_v[pl.ds(p * PIECE + g * 16, 16)]
            for l in range(16):
                i = g * 16 + l
                aval = adis16[l]
                for jj in range(HD // 16):
                    stage_v[i, pl.ds(jj * 16, 16)] = (
                        stage_v[i, pl.ds(jj * 16, 16)] * aval)
            return ()

        lax.fori_loop(0, PIECE // 16, groupscale, ())
        pltpu.sync_copy(stage_v, out_hbm.at[pl.ds(r0, PIECE), c])
        return ()

    lax.fori_loop(0, npieces, drain, ())


def kernel(x, edge_index, alpha):
    row = edge_index[0].astype(jnp.int32)
    col = edge_index[1].astype(jnp.int32)
    pad_n = E_PAD - E
    pad_row = jnp.arange(pad_n, dtype=jnp.int32) % N_NODES
    pad_col = N_NODES + jnp.arange(pad_n, dtype=jnp.int32) % (N_PAD - N_NODES)
    row3 = jnp.concatenate([row, pad_row]).reshape(NS, NCH, C)
    col3 = jnp.concatenate([col, pad_col]).reshape(NS, NCH, C)
    alpha16 = jnp.broadcast_to(alpha.astype(jnp.float32), (16,))

    adis, xs = _norm_kernel(col3, alpha16, x)
    out = _agg_kernel(xs, row3, col3, adis)
    return out.reshape(N_NODES, D)


# speedup vs baseline: 1.0996x; 1.0996x over previous
import functools

import jax
import jax.numpy as jnp
from jax import lax
from jax.experimental import pallas as pl
from jax.experimental.pallas import tpu as pltpu
from jax.experimental.pallas import tpu_sc as plsc

N_NODES = 10000
D = 128
HD = D // 2
E = 320000

NC = 2
NS = 16
C = 128
NCH = 160
ECH = E // C
NCH_LAST = ECH - (NS - 1) * NCH
N_PAD = 10240
RPT = N_PAD // NS
PIECE = 80
NPIECE = RPT // PIECE
NPIECE_LAST = (N_NODES - (NS - 1) * RPT) // PIECE

_sc_mesh = plsc.VectorSubcoreMesh(core_axis_name="c", subcore_axis_name="s")
_sc_params = pltpu.CompilerParams(use_tc_tiling_on_sc=False,
                                  needs_layout_passes=False)


def _rsqrt16(d):
    i = plsc.bitcast(d, jnp.int32)
    y = plsc.bitcast(jnp.int32(0x5F3759DF) - (i >> 1), jnp.float32)
    for _ in range(3):
        y = y * (1.5 - 0.5 * d * y * y)
    return jnp.where(d > 0.5, y, 0.0)


def _stage_edges(e3_hbm, which, s, dst_v):
    @pl.when(s < NS - 1)
    def _full():
        pltpu.sync_copy(e3_hbm.at[which, pl.ds(s * NCH, NCH)], dst_v)

    @pl.when(s == NS - 1)
    def _last():
        pltpu.sync_copy(e3_hbm.at[which, pl.ds((NS - 1) * NCH, NCH_LAST)],
                        dst_v.at[pl.ds(0, NCH_LAST)])


@functools.partial(
    pl.kernel,
    out_type=(
        jax.ShapeDtypeStruct((NC, N_PAD), jnp.float32),
        jax.ShapeDtypeStruct((NC, N_NODES, HD), jnp.float32),
    ),
    mesh=_sc_mesh,
    scratch_types=[
        pltpu.VMEM((NCH, C), jnp.int32),
        pltpu.VMEM((C,), jnp.float32),
        pltpu.VMEM((RPT,), jnp.float32),
        pltpu.VMEM((RPT,), jnp.float32),
        pltpu.VMEM((RPT,), jnp.float32),
        pltpu.VMEM((16,), jnp.float32),
        pltpu.VMEM((PIECE, D), jnp.float32),
        pltpu.VMEM((PIECE, HD), jnp.float32),
        pltpu.VMEM_SHARED((N_PAD,), jnp.float32),
        pltpu.SemaphoreType.DMA,
    ],
    compiler_params=_sc_params,
)
def _norm_kernel(e3_hbm, alpha_hbm, x_hbm, adis_hbm, xs_hbm,
                 col_v, ones_v, deg_v, dis_v, adis_v, alpha_v,
                 xp_v, xsp_v, deg_sh, ssem):
    c = lax.axis_index("c")
    s = lax.axis_index("s")
    nch = jnp.where(s == NS - 1, NCH_LAST, NCH)
    one16 = jnp.ones((16,), jnp.float32)
    zero16 = jnp.zeros((16,), jnp.float32)
    for i in range(C // 16):
        ones_v[pl.ds(i * 16, 16)] = one16
    for i in range(RPT // 16):
        deg_v[pl.ds(i * 16, 16)] = zero16
    pltpu.sync_copy(deg_v, deg_sh.at[pl.ds(s * RPT, RPT)])
    _stage_edges(e3_hbm, 1, s, col_v)
    pltpu.sync_copy(alpha_hbm, alpha_v)
    plsc.subcore_barrier()

    def hist(r, _):
        for b in range(4):
            pltpu.async_copy(ones_v, deg_sh.at[col_v.at[4 * r + b]], ssem,
                             add=True)
        for b in range(4):
            pltpu.make_async_copy(ones_v, deg_sh.at[col_v.at[4 * r + b]],
                                  ssem).wait()
        return ()

    lax.fori_loop(0, nch // 4, hist, ())
    plsc.subcore_barrier()

    pltpu.sync_copy(deg_sh.at[pl.ds(s * RPT, RPT)], deg_v)
    av = alpha_v[...]
    for i in range(RPT // 16):
        dis16 = _rsqrt16(deg_v[pl.ds(i * 16, 16)])
        dis_v[pl.ds(i * 16, 16)] = dis16
        adis_v[pl.ds(i * 16, 16)] = dis16 * av
    pltpu.sync_copy(adis_v, adis_hbm.at[c, pl.ds(s * RPT, RPT)])

    base_col = c * HD
    npieces = jnp.where(s == NS - 1, NPIECE_LAST, NPIECE)

    def prescale(p, _):
        r0 = s * RPT + p * PIECE
        pltpu.sync_copy(x_hbm.at[pl.ds(r0, PIECE)], xp_v)

        def groupscale(g, _):
            dis16 = dis_v[pl.ds(p * PIECE + g * 16, 16)]
            for l in range(16):
                i = g * 16 + l
                dval = dis16[l]
                for jj in range(HD // 16):
                    xsp_v[i, pl.ds(jj * 16, 16)] = (
                        xp_v[i, pl.ds(base_col + jj * 16, 16)] * dval)
            return ()

        lax.fori_loop(0, PIECE // 16, groupscale, ())
        pltpu.sync_copy(xsp_v, xs_hbm.at[c, pl.ds(r0, PIECE)])
        return ()

    lax.fori_loop(0, npieces, prescale, ())


@functools.partial(
    pl.kernel,
    out_type=jax.ShapeDtypeStruct((N_NODES, NC, HD), jnp.float32),
    mesh=_sc_mesh,
    scratch_types=[
        pltpu.VMEM((NCH, C), jnp.int32),
        pltpu.VMEM((NCH, C), jnp.int32),
        pltpu.VMEM((4, C, HD), jnp.float32),
        pltpu.VMEM((C, HD), jnp.float32),
        pltpu.VMEM((RPT,), jnp.float32),
        pltpu.VMEM((PIECE, HD), jnp.float32),
        pltpu.VMEM_SHARED((N_PAD, HD), jnp.float32),
        pltpu.SemaphoreType.DMA,
        pltpu.SemaphoreType.DMA,
    ],
    compiler_params=_sc_params,
)
def _agg_kernel(xs_hbm, e3_hbm, adis_hbm, out_hbm,
                row_v, col_v, msg_v, zero_v, adis_v, stage_v, acc_sh,
                gsem, ssem):
    c = lax.axis_index("c")
    s = lax.axis_index("s")
    nch = jnp.where(s == NS - 1, NCH_LAST, NCH)
    zero16 = jnp.zeros((16,), jnp.float32)

    def zbody(i, _):
        for jj in range(HD // 16):
            zero_v[i, pl.ds(jj * 16, 16)] = zero16
        return ()

    lax.fori_loop(0, C, zbody, ())
    for piece in range(RPT // C):
        pltpu.sync_copy(zero_v, acc_sh.at[pl.ds(s * RPT + piece * C, C)])
    _stage_edges(e3_hbm, 0, s, row_v)
    _stage_edges(e3_hbm, 1, s, col_v)
    pltpu.sync_copy(adis_hbm.at[c, pl.ds(s * RPT, RPT)], adis_v)
    plsc.subcore_barrier()

    xsc = xs_hbm.at[c]

    for b in range(3):
        pltpu.async_copy(xsc.at[row_v.at[b]], msg_v.at[b], gsem)

    def round_body(r, _):
        for b in range(4):
            k = 4 * r + b
            pltpu.make_async_copy(xsc.at[row_v.at[k]], msg_v.at[b], gsem).wait()
            pltpu.async_copy(msg_v.at[b], acc_sh.at[col_v.at[k]], ssem,
                             add=True)

            @pl.when(k >= 1)
            def _release_prev():
                pltpu.make_async_copy(msg_v.at[(b + 3) % 4],
                                      acc_sh.at[col_v.at[k - 1]], ssem).wait()

            @pl.when(k + 3 < nch)
            def _launch_next():
                pltpu.async_copy(xsc.at[row_v.at[k + 3]],
                                 msg_v.at[(b + 3) % 4], gsem)
        return ()

    lax.fori_loop(0, nch // 4, round_body, ())
    pltpu.make_async_copy(msg_v.at[3], acc_sh.at[col_v.at[nch - 1]],
                          ssem).wait()
    plsc.subcore_barrier()

    npieces = jnp.where(s == NS - 1, NPIECE_LAST, NPIECE)

    def drain(p, _):
        r0 = s * RPT + p * PIECE
        pltpu.sync_copy(acc_sh.at[pl.ds(r0, PIECE)], stage_v)

        def groupscale(g, _):
            adis16 = adis_v[pl.ds(p * PIECE + g * 16, 16)]
            for l in range(16):
                i = g * 16 + l
                aval = adis16[l]
                for jj in range(HD // 16):
                    stage_v[i, pl.ds(jj * 16, 16)] = (
                        stage_v[i, pl.ds(jj * 16, 16)] * aval)
            return ()

        lax.fori_loop(0, PIECE // 16, groupscale, ())
        pltpu.sync_copy(stage_v, out_hbm.at[pl.ds(r0, PIECE), c])
        return ()

    lax.fori_loop(0, npieces, drain, ())


def kernel(x, edge_index, alpha):
    e3 = edge_index.astype(jnp.int32).reshape(2, ECH, C)
    alpha16 = jnp.broadcast_to(alpha.astype(jnp.float32), (16,))
    adis, xs = _norm_kernel(e3, alpha16, x)
    out = _agg_kernel(xs, e3, adis)
    return out.reshape(N_NODES, D)


# speedup vs baseline: 1.3728x; 1.2484x over previous
import functools

import jax
import jax.numpy as jnp
from jax import lax
from jax.experimental import pallas as pl
from jax.experimental.pallas import tpu as pltpu
from jax.experimental.pallas import tpu_sc as plsc

N_NODES = 10000
D = 128
HD = D // 2
E = 320000

NC = 2
NS = 16
C = 128
NCH = 160
ECH = E // C
NCH_LAST = ECH - (NS - 1) * NCH
N_PAD = 10240
RPT = N_PAD // NS
PIECE = 80
NPIECE = RPT // PIECE
NPIECE_LAST = (N_NODES - (NS - 1) * RPT) // PIECE

_sc_mesh = plsc.VectorSubcoreMesh(core_axis_name="c", subcore_axis_name="s")
_sc_params = pltpu.CompilerParams(use_tc_tiling_on_sc=False,
                                  needs_layout_passes=False)


def _rsqrt16(d):
    i = plsc.bitcast(d, jnp.int32)
    y = plsc.bitcast(jnp.int32(0x5F3759DF) - (i >> 1), jnp.float32)
    for _ in range(3):
        y = y * (1.5 - 0.5 * d * y * y)
    return jnp.where(d > 0.5, y, 0.0)


def _stage_edges(e3_hbm, which, s, dst_v):
    @pl.when(s < NS - 1)
    def _full():
        pltpu.sync_copy(e3_hbm.at[which, pl.ds(s * NCH, NCH)], dst_v)

    @pl.when(s == NS - 1)
    def _last():
        pltpu.sync_copy(e3_hbm.at[which, pl.ds((NS - 1) * NCH, NCH_LAST)],
                        dst_v.at[pl.ds(0, NCH_LAST)])


@functools.partial(
    pl.kernel,
    out_type=(
        jax.ShapeDtypeStruct((NC, N_PAD), jnp.float32),
        jax.ShapeDtypeStruct((NC, N_NODES, HD), jnp.float32),
    ),
    mesh=_sc_mesh,
    scratch_types=[
        pltpu.VMEM((NCH, C), jnp.int32),
        pltpu.VMEM((C,), jnp.float32),
        pltpu.VMEM((RPT,), jnp.float32),
        pltpu.VMEM((RPT,), jnp.float32),
        pltpu.VMEM((RPT,), jnp.float32),
        pltpu.VMEM((16,), jnp.float32),
        pltpu.VMEM((PIECE, D), jnp.float32),
        pltpu.VMEM((PIECE, HD), jnp.float32),
        pltpu.VMEM_SHARED((N_PAD,), jnp.float32),
        pltpu.SemaphoreType.DMA,
    ],
    compiler_params=_sc_params,
)
def _norm_kernel(e3_hbm, alpha_hbm, x_hbm, adis_hbm, xs_hbm,
                 col_v, ones_v, deg_v, dis_v, adis_v, alpha_v,
                 xp_v, xsp_v, deg_sh, ssem):
    c = lax.axis_index("c")
    s = lax.axis_index("s")
    nch = jnp.where(s == NS - 1, NCH_LAST, NCH)
    one16 = jnp.ones((16,), jnp.float32)
    zero16 = jnp.zeros((16,), jnp.float32)
    for i in range(C // 16):
        ones_v[pl.ds(i * 16, 16)] = one16
    for i in range(RPT // 16):
        deg_v[pl.ds(i * 16, 16)] = zero16
    pltpu.sync_copy(deg_v, deg_sh.at[pl.ds(s * RPT, RPT)])
    _stage_edges(e3_hbm, 1, s, col_v)
    pltpu.sync_copy(alpha_hbm, alpha_v)
    plsc.subcore_barrier()

    def hist(r, _):
        for b in range(4):
            pltpu.async_copy(ones_v, deg_sh.at[col_v.at[4 * r + b]], ssem,
                             add=True)
        for b in range(4):
            pltpu.make_async_copy(ones_v, deg_sh.at[col_v.at[4 * r + b]],
                                  ssem).wait()
        return ()

    lax.fori_loop(0, nch // 4, hist, ())
    plsc.subcore_barrier()

    pltpu.sync_copy(deg_sh.at[pl.ds(s * RPT, RPT)], deg_v)
    av = alpha_v[...]
    for i in range(RPT // 16):
        dis16 = _rsqrt16(deg_v[pl.ds(i * 16, 16)])
        dis_v[pl.ds(i * 16, 16)] = dis16
        adis_v[pl.ds(i * 16, 16)] = dis16 * av
    pltpu.sync_copy(adis_v, adis_hbm.at[c, pl.ds(s * RPT, RPT)])

    base_col = c * HD
    npieces = jnp.where(s == NS - 1, NPIECE_LAST, NPIECE)

    def prescale(p, _):
        r0 = s * RPT + p * PIECE
        pltpu.sync_copy(x_hbm.at[pl.ds(r0, PIECE)], xp_v)

        def groupscale(g, _):
            dis16 = dis_v[pl.ds(p * PIECE + g * 16, 16)]
            for l in range(16):
                i = g * 16 + l
                dval = dis16[l]
                for jj in range(HD // 16):
                    xsp_v[i, pl.ds(jj * 16, 16)] = (
                        xp_v[i, pl.ds(base_col + jj * 16, 16)] * dval)
            return ()

        lax.fori_loop(0, PIECE // 16, groupscale, ())
        pltpu.sync_copy(xsp_v, xs_hbm.at[c, pl.ds(r0, PIECE)])
        return ()

    lax.fori_loop(0, npieces, prescale, ())


@functools.partial(
    pl.kernel,
    out_type=jax.ShapeDtypeStruct((N_NODES, D), jnp.float32),
    mesh=_sc_mesh,
    scratch_types=[
        pltpu.VMEM((NCH, C), jnp.int32),
        pltpu.VMEM((NCH, C), jnp.int32),
        pltpu.VMEM((4, C, HD), jnp.float32),
        pltpu.VMEM((C, HD), jnp.float32),
        pltpu.VMEM((RPT,), jnp.float32),
        pltpu.VMEM((PIECE, HD), jnp.float32),
        pltpu.VMEM_SHARED((N_PAD, HD), jnp.float32),
        pltpu.SemaphoreType.DMA,
        pltpu.SemaphoreType.DMA,
    ],
    compiler_params=_sc_params,
)
def _agg_kernel(xs_hbm, e3_hbm, adis_hbm, out_hbm,
                row_v, col_v, msg_v, zero_v, adis_v, stage_v, acc_sh,
                gsem, ssem):
    c = lax.axis_index("c")
    s = lax.axis_index("s")
    nch = jnp.where(s == NS - 1, NCH_LAST, NCH)
    zero16 = jnp.zeros((16,), jnp.float32)

    def zbody(i, _):
        for jj in range(HD // 16):
            zero_v[i, pl.ds(jj * 16, 16)] = zero16
        return ()

    lax.fori_loop(0, C, zbody, ())
    for piece in range(RPT // C):
        pltpu.sync_copy(zero_v, acc_sh.at[pl.ds(s * RPT + piece * C, C)])
    _stage_edges(e3_hbm, 0, s, row_v)
    _stage_edges(e3_hbm, 1, s, col_v)
    pltpu.sync_copy(adis_hbm.at[c, pl.ds(s * RPT, RPT)], adis_v)
    plsc.subcore_barrier()

    xsc = xs_hbm.at[c]

    for b in range(3):
        pltpu.async_copy(xsc.at[row_v.at[b]], msg_v.at[b], gsem)

    def round_body(r, _):
        for b in range(4):
            k = 4 * r + b
            pltpu.make_async_copy(xsc.at[row_v.at[k]], msg_v.at[b], gsem).wait()
            pltpu.async_copy(msg_v.at[b], acc_sh.at[col_v.at[k]], ssem,
                             add=True)

            @pl.when(k >= 1)
            def _release_prev():
                pltpu.make_async_copy(msg_v.at[(b + 3) % 4],
                                      acc_sh.at[col_v.at[k - 1]], ssem).wait()

            @pl.when(k + 3 < nch)
            def _launch_next():
                pltpu.async_copy(xsc.at[row_v.at[k + 3]],
                                 msg_v.at[(b + 3) % 4], gsem)
        return ()

    lax.fori_loop(0, nch // 4, round_body, ())
    pltpu.make_async_copy(msg_v.at[3], acc_sh.at[col_v.at[nch - 1]],
                          ssem).wait()
    plsc.subcore_barrier()

    npieces = jnp.where(s == NS - 1, NPIECE_LAST, NPIECE)

    def drain(p, _):
        r0 = s * RPT + p * PIECE
        pltpu.sync_copy(acc_sh.at[pl.ds(r0, PIECE)], stage_v)

        def groupscale(g, _):
            adis16 = adis_v[pl.ds(p * PIECE + g * 16, 16)]
            for l in range(16):
                i = g * 16 + l
                aval = adis16[l]
                for jj in range(HD // 16):
                    stage_v[i, pl.ds(jj * 16, 16)] = (
                        stage_v[i, pl.ds(jj * 16, 16)] * aval)
            return ()

        lax.fori_loop(0, PIECE // 16, groupscale, ())

        @pl.when(c == 0)
        def _left():
            pltpu.sync_copy(stage_v, out_hbm.at[pl.ds(r0, PIECE), pl.ds(0, HD)])

        @pl.when(c == 1)
        def _right():
            pltpu.sync_copy(stage_v, out_hbm.at[pl.ds(r0, PIECE), pl.ds(HD, HD)])

        return ()

    lax.fori_loop(0, npieces, drain, ())


def kernel(x, edge_index, alpha):
    e3 = edge_index.astype(jnp.int32).reshape(2, ECH, C)
    alpha16 = jnp.broadcast_to(alpha.astype(jnp.float32), (16,))
    adis, xs = _norm_kernel(e3, alpha16, x)
    return _agg_kernel(xs, e3, adis)


# speedup vs baseline: 1.4631x; 1.0658x over previous
import functools

import jax
import jax.numpy as jnp
from jax import lax
from jax.experimental import pallas as pl
from jax.experimental.pallas import tpu as pltpu
from jax.experimental.pallas import tpu_sc as plsc

N_NODES = 10000
D = 128
HD = D // 2
E = 320000

NC = 2
NS = 16
C = 128
NCH = 160
ECH = E // C
NCH_LAST = ECH - (NS - 1) * NCH
N_PAD = 10240
RPT = N_PAD // NS
PIECE = 80
NPIECE = RPT // PIECE
NPIECE_LAST = (N_NODES - (NS - 1) * RPT) // PIECE

_sc_mesh = plsc.VectorSubcoreMesh(core_axis_name="c", subcore_axis_name="s")
_sc_params = pltpu.CompilerParams(use_tc_tiling_on_sc=False,
                                  needs_layout_passes=False)


def _rsqrt16(d):
    i = plsc.bitcast(d, jnp.int32)
    y = plsc.bitcast(jnp.int32(0x5F3759DF) - (i >> 1), jnp.float32)
    for _ in range(3):
        y = y * (1.5 - 0.5 * d * y * y)
    return jnp.where(d > 0.5, y, 0.0)


def _stage_edges(e3_hbm, which, s, dst_v):
    @pl.when(s < NS - 1)
    def _full():
        pltpu.sync_copy(e3_hbm.at[which, pl.ds(s * NCH, NCH)], dst_v)

    @pl.when(s == NS - 1)
    def _last():
        pltpu.sync_copy(e3_hbm.at[which, pl.ds((NS - 1) * NCH, NCH_LAST)],
                        dst_v.at[pl.ds(0, NCH_LAST)])


@functools.partial(
    pl.kernel,
    out_type=(
        jax.ShapeDtypeStruct((NC, N_PAD), jnp.float32),
        jax.ShapeDtypeStruct((NC, N_NODES, HD), jnp.float32),
    ),
    mesh=_sc_mesh,
    scratch_types=[
        pltpu.VMEM((NCH, C), jnp.int32),
        pltpu.VMEM((C,), jnp.float32),
        pltpu.VMEM((RPT,), jnp.float32),
        pltpu.VMEM((RPT,), jnp.float32),
        pltpu.VMEM((RPT,), jnp.float32),
        pltpu.VMEM((16,), jnp.float32),
        pltpu.VMEM((2, PIECE, D), jnp.float32),
        pltpu.VMEM((PIECE, HD), jnp.float32),
        pltpu.VMEM_SHARED((N_PAD,), jnp.float32),
        pltpu.SemaphoreType.DMA,
        pltpu.SemaphoreType.DMA,
    ],
    compiler_params=_sc_params,
)
def _norm_kernel(e3_hbm, alpha_hbm, x_hbm, adis_hbm, xs_hbm,
                 col_v, ones_v, deg_v, dis_v, adis_v, alpha_v,
                 xp_v, xsp_v, deg_sh, ssem, gsem):
    c = lax.axis_index("c")
    s = lax.axis_index("s")
    nch = jnp.where(s == NS - 1, NCH_LAST, NCH)
    one16 = jnp.ones((16,), jnp.float32)
    zero16 = jnp.zeros((16,), jnp.float32)
    for i in range(C // 16):
        ones_v[pl.ds(i * 16, 16)] = one16
    for i in range(RPT // 16):
        deg_v[pl.ds(i * 16, 16)] = zero16
    pltpu.sync_copy(deg_v, deg_sh.at[pl.ds(s * RPT, RPT)])
    _stage_edges(e3_hbm, 1, s, col_v)
    pltpu.sync_copy(alpha_hbm, alpha_v)
    plsc.subcore_barrier()

    def hist(r, _):
        for b in range(10):
            pltpu.async_copy(ones_v, deg_sh.at[col_v.at[10 * r + b]], ssem,
                             add=True)
        for b in range(10):
            pltpu.make_async_copy(ones_v, deg_sh.at[col_v.at[10 * r + b]],
                                  ssem).wait()
        return ()

    lax.fori_loop(0, nch // 10, hist, ())
    plsc.subcore_barrier()

    pltpu.sync_copy(deg_sh.at[pl.ds(s * RPT, RPT)], deg_v)
    av = alpha_v[...]
    for i in range(RPT // 16):
        dis16 = _rsqrt16(deg_v[pl.ds(i * 16, 16)])
        dis_v[pl.ds(i * 16, 16)] = dis16
        adis_v[pl.ds(i * 16, 16)] = dis16 * av
    pltpu.sync_copy(adis_v, adis_hbm.at[c, pl.ds(s * RPT, RPT)])

    base_col = c * HD
    row_base = s * RPT
    pltpu.async_copy(x_hbm.at[pl.ds(row_base, PIECE)], xp_v.at[0], gsem)

    def prescale_round(r, _):
        for b in range(2):
            p = 2 * r + b
            r0 = row_base + p * PIECE

            @pl.when(r0 + PIECE <= N_NODES)
            def _piece():
                pltpu.make_async_copy(x_hbm.at[pl.ds(r0, PIECE)],
                                      xp_v.at[b], gsem).wait()

                @pl.when(r0 + 2 * PIECE <= N_NODES)
                def _prefetch():
                    pltpu.async_copy(x_hbm.at[pl.ds(r0 + PIECE, PIECE)],
                                     xp_v.at[1 - b], gsem)

                def groupscale(g, _):
                    dis16 = dis_v[pl.ds(p * PIECE + g * 16, 16)]
                    for l in range(16):
                        i = g * 16 + l
                        dval = dis16[l]
                        for jj in range(HD // 16):
                            xsp_v[i, pl.ds(jj * 16, 16)] = (
                                xp_v[b, i, pl.ds(base_col + jj * 16, 16)]
                                * dval)
                    return ()

                lax.fori_loop(0, PIECE // 16, groupscale, ())
                pltpu.sync_copy(xsp_v, xs_hbm.at[c, pl.ds(r0, PIECE)])

        return ()

    lax.fori_loop(0, NPIECE // 2, prescale_round, ())


@functools.partial(
    pl.kernel,
    out_type=jax.ShapeDtypeStruct((N_NODES, D), jnp.float32),
    mesh=_sc_mesh,
    scratch_types=[
        pltpu.VMEM((NCH, C), jnp.int32),
        pltpu.VMEM((NCH, C), jnp.int32),
        pltpu.VMEM((4, C, HD), jnp.float32),
        pltpu.VMEM((C, HD), jnp.float32),
        pltpu.VMEM((RPT,), jnp.float32),
        pltpu.VMEM((PIECE, HD), jnp.float32),
        pltpu.VMEM_SHARED((N_PAD, HD), jnp.float32),
        pltpu.SemaphoreType.DMA,
        pltpu.SemaphoreType.DMA,
    ],
    compiler_params=_sc_params,
)
def _agg_kernel(xs_hbm, e3_hbm, adis_hbm, out_hbm,
                row_v, col_v, msg_v, zero_v, adis_v, stage_v, acc_sh,
                gsem, ssem):
    c = lax.axis_index("c")
    s = lax.axis_index("s")
    nch = jnp.where(s == NS - 1, NCH_LAST, NCH)
    zero16 = jnp.zeros((16,), jnp.float32)

    def zbody(i, _):
        for jj in range(HD // 16):
            zero_v[i, pl.ds(jj * 16, 16)] = zero16
        return ()

    lax.fori_loop(0, C, zbody, ())
    for piece in range(RPT // C):
        pltpu.sync_copy(zero_v, acc_sh.at[pl.ds(s * RPT + piece * C, C)])
    _stage_edges(e3_hbm, 0, s, row_v)
    _stage_edges(e3_hbm, 1, s, col_v)
    pltpu.sync_copy(adis_hbm.at[c, pl.ds(s * RPT, RPT)], adis_v)
    plsc.subcore_barrier()

    xsc = xs_hbm.at[c]

    for b in range(3):
        pltpu.async_copy(xsc.at[row_v.at[b]], msg_v.at[b], gsem)

    def round_body(r, _):
        for b in range(4):
            k = 4 * r + b
            pltpu.make_async_copy(xsc.at[row_v.at[k]], msg_v.at[b], gsem).wait()
            pltpu.async_copy(msg_v.at[b], acc_sh.at[col_v.at[k]], ssem,
                             add=True)

            @pl.when(k >= 1)
            def _release_prev():
                pltpu.make_async_copy(msg_v.at[(b + 3) % 4],
                                      acc_sh.at[col_v.at[k - 1]], ssem).wait()

            @pl.when(k + 3 < nch)
            def _launch_next():
                pltpu.async_copy(xsc.at[row_v.at[k + 3]],
                                 msg_v.at[(b + 3) % 4], gsem)
        return ()

    lax.fori_loop(0, nch // 4, round_body, ())
    pltpu.make_async_copy(msg_v.at[3], acc_sh.at[col_v.at[nch - 1]],
                          ssem).wait()
    plsc.subcore_barrier()

    npieces = jnp.where(s == NS - 1, NPIECE_LAST, NPIECE)

    def drain(p, _):
        r0 = s * RPT + p * PIECE
        pltpu.sync_copy(acc_sh.at[pl.ds(r0, PIECE)], stage_v)

        def groupscale(g, _):
            adis16 = adis_v[pl.ds(p * PIECE + g * 16, 16)]
            for l in range(16):
                i = g * 16 + l
                aval = adis16[l]
                for jj in range(HD // 16):
                    stage_v[i, pl.ds(jj * 16, 16)] = (
                        stage_v[i, pl.ds(jj * 16, 16)] * aval)
            return ()

        lax.fori_loop(0, PIECE // 16, groupscale, ())

        @pl.when(c == 0)
        def _left():
            pltpu.sync_copy(stage_v, out_hbm.at[pl.ds(r0, PIECE), pl.ds(0, HD)])

        @pl.when(c == 1)
        def _right():
            pltpu.sync_copy(stage_v, out_hbm.at[pl.ds(r0, PIECE), pl.ds(HD, HD)])

        return ()

    lax.fori_loop(0, npieces, drain, ())


def kernel(x, edge_index, alpha):
    e3 = edge_index.astype(jnp.int32).reshape(2, ECH, C)
    alpha16 = jnp.broadcast_to(alpha.astype(jnp.float32), (16,))
    adis, xs = _norm_kernel(e3, alpha16, x)
    return _agg_kernel(xs, e3, adis)
